# baseline (speedup 1.0000x reference)
import functools

import jax
import jax.numpy as jnp
from jax import lax
from jax.experimental import pallas as pl
from jax.experimental.pallas import tpu as pltpu
from jax.experimental.pallas import tpu_sc as plsc

BB, TT, DD, KK = 16, 576, 256, 8192
NN = BB * TT
COMMIT_W = 0.25


BM = 512
BK = 1024
NM = NN // BM
NK = KK // BK


def _argmin_body(zn_ref, cn_ref, x_ref, c_ref, idx_ref, rmin_ref, ridx_ref):
    k = pl.program_id(1)

    @pl.when(k == 0)
    def _init():
        rmin_ref[...] = jnp.full((BM, 1), jnp.inf, jnp.float32)
        ridx_ref[...] = jnp.zeros((BM, 1), jnp.int32)

    dot = lax.dot_general(x_ref[...], c_ref[...], (((1,), (1,)), ((), ())),
                          preferred_element_type=jnp.float32)
    dist = (zn_ref[...] - 2.0 * dot) + cn_ref[...]
    bmin = jnp.min(dist, axis=1, keepdims=True)
    cols = lax.broadcasted_iota(jnp.int32, (BM, BK), 1) + k * BK
    cand = jnp.where(dist == bmin, cols, jnp.int32(2 ** 30))
    bidx = jnp.min(cand, axis=1, keepdims=True)
    better = bmin < rmin_ref[...]
    rmin_ref[...] = jnp.where(better, bmin, rmin_ref[...])
    ridx_ref[...] = jnp.where(better, bidx, ridx_ref[...])

    @pl.when(k == NK - 1)
    def _fin():
        idx_ref[...] = ridx_ref[...]


def _argmin_call(zf, codebook, zn, cn):
    return pl.pallas_call(
        _argmin_body,
        grid=(NM, NK),
        in_specs=[
            pl.BlockSpec((BM, 1), lambda m, k: (m, 0)),
            pl.BlockSpec((1, BK), lambda m, k: (0, k)),
            pl.BlockSpec((BM, DD), lambda m, k: (m, 0)),
            pl.BlockSpec((BK, DD), lambda m, k: (k, 0)),
        ],
        out_specs=pl.BlockSpec((BM, 1), lambda m, k: (m, 0)),
        out_shape=jax.ShapeDtypeStruct((NN, 1), jnp.int32),
        scratch_shapes=[
            pltpu.VMEM((BM, 1), jnp.float32),
            pltpu.VMEM((BM, 1), jnp.int32),
        ],
    )(zn, cn, zf, codebook)


SC_CORES = 2
SC_SUB = 16
SC_W = SC_CORES * SC_SUB
TPW = NN // SC_W
CH = 96
NCH = TPW // CH
DV = DD // 16


def _sc_body(idx_hbm, z_hbm, cb_hbm, zq_hbm, res_hbm, hist_hbm, ss_hbm,
             idx_v, rows_v, z_v, r_v, acc_v, ones_v, zer_v, hist_sh, sem):
    cid = lax.axis_index("c")
    sid = lax.axis_index("s")
    wid = sid * SC_CORES + cid
    base = wid * TPW
    bi = wid // 2
    half = (wid % 2) * TPW

    for ci in range(NCH):
        pltpu.sync_copy(idx_hbm.at[pl.ds(base + ci * CH, CH)], idx_v.at[ci])

    ones16 = jnp.ones((16,), jnp.float32)
    for j in range(CH // 16):
        ones_v[pl.ds(j * 16, 16)] = ones16

    @pl.when(sid == 0)
    def _zero_hist():
        zero16 = jnp.zeros((16,), jnp.float32)

        def zbody(i, carry):
            zer_v[pl.ds(i * 16, 16)] = zero16
            return carry

        lax.fori_loop(0, KK // 16, zbody, 0)
        pltpu.sync_copy(zer_v, hist_sh)

    plsc.subcore_barrier()

    for ci in range(NCH):
        pltpu.sync_copy(ones_v, hist_sh.at[idx_v.at[ci]], add=True)

    acc = jnp.zeros((16,), jnp.float32)
    for ci in range(NCH):
        row = half + ci * CH
        pltpu.sync_copy(z_hbm.at[bi, pl.ds(row, CH)], z_v)
        pltpu.async_copy(cb_hbm.at[idx_v.at[ci]], rows_v, sem).wait()

        def tbody(t, a):
            for d in range(DV):
                q = rows_v[t, pl.ds(d * 16, 16)]
                zz = z_v[t, pl.ds(d * 16, 16)]
                r = zz - q
                r_v[t, pl.ds(d * 16, 16)] = r
                a = a + r * r
            return a

        acc = lax.fori_loop(0, CH, tbody, acc)
        pltpu.sync_copy(rows_v, zq_hbm.at[bi, pl.ds(row, CH)])
        pltpu.sync_copy(r_v, res_hbm.at[bi, pl.ds(row, CH)])

    acc_v[...] = acc
    pltpu.sync_copy(acc_v, ss_hbm.at[wid])

    plsc.subcore_barrier()

    @pl.when(sid == 0)
    def _emit_hist():
        pltpu.sync_copy(hist_sh, hist_hbm.at[cid])


def _sc_call(idx, z3d, codebook):
    mesh = plsc.VectorSubcoreMesh(core_axis_name="c", subcore_axis_name="s")
    fn = pl.kernel(
        _sc_body,
        mesh=mesh,
        out_type=[
            jax.ShapeDtypeStruct((BB, TT, DD), jnp.float32),
            jax.ShapeDtypeStruct((BB, TT, DD), jnp.float32),
            jax.ShapeDtypeStruct((SC_CORES, KK), jnp.float32),
            jax.ShapeDtypeStruct((SC_W, 16), jnp.float32),
        ],
        scratch_types=[
            pltpu.VMEM((NCH, CH), jnp.int32),
            pltpu.VMEM((CH, DD), jnp.float32),
            pltpu.VMEM((CH, DD), jnp.float32),
            pltpu.VMEM((CH, DD), jnp.float32),
            pltpu.VMEM((16,), jnp.float32),
            pltpu.VMEM((CH,), jnp.float32),
            pltpu.VMEM((KK,), jnp.float32),
            pltpu.VMEM_SHARED((KK,), jnp.float32),
            pltpu.SemaphoreType.DMA,
        ],
    )
    return fn(idx, z3d, codebook)


def _stats_body(hist_ref, ss_ref, vq_ref, perp_ref, used_ref, frac_ref):
    counts = hist_ref[0:1, :] + hist_ref[1:2, :]
    total = jnp.maximum(jnp.sum(counts), 1.0)
    p = counts / total
    logp = jnp.log(p + 1e-10)
    ent = -jnp.sum(p * logp)
    perp = jnp.exp(ent)
    used = jnp.sum((counts > 0.0).astype(jnp.float32))
    ssum = jnp.sum(ss_ref[...])
    lm = ssum * (1.0 / float(NN * DD))
    vq_ref[0, 0] = lm + COMMIT_W * lm
    perp_ref[0, 0] = perp
    used_ref[0, 0] = used
    frac_ref[0, 0] = used / float(KK)


def _stats_call(hist, ss):
    smem = pl.BlockSpec(memory_space=pltpu.SMEM)
    return pl.pallas_call(
        _stats_body,
        out_specs=[smem, smem, smem, smem],
        out_shape=[jax.ShapeDtypeStruct((1, 1), jnp.float32)] * 4,
    )(hist, ss)


def kernel(z, codebook):
    zf = z.reshape(NN, DD)
    zn = jnp.sum(zf ** 2, axis=1, keepdims=True)
    cn = jnp.sum(codebook ** 2, axis=1)[None, :]
    idx = _argmin_call(zf, codebook, zn, cn).reshape(NN)
    zq, res, hist, ss = _sc_call(idx, z, codebook)
    vq, perp, used, frac = _stats_call(hist, ss)
    return (res, vq[0, 0], zq, perp[0, 0], used[0, 0], frac[0, 0])
